# baseline (speedup 1.0000x reference)
import functools

import jax
import jax.numpy as jnp
from jax import lax
from jax.experimental import pallas as pl
from jax.experimental.pallas import tpu as pltpu
from jax.experimental.pallas import tpu_sc as plsc

_NC = 2
_NS = 16
_NW = _NC * _NS
_L = 16

_MAGIC2 = 0x5F375F37


def _dist_bf16(d2):
    i = plsc.bitcast(d2, jnp.int32)
    seed = jnp.full((_L,), _MAGIC2, jnp.int32) - ((i >> 1) & 0x7FFF7FFF)
    y = plsc.bitcast(seed, jnp.bfloat16)
    t = d2 * y
    return t * (1.5 - 0.5 * (t * y))


def _make_sc_kernel(B, N, T):
    BN = B * N
    rows_per = BN // _NW
    CH = 64
    n_chunks = rows_per // CH
    pairs = n_chunks // 2
    CE = CH * T

    mesh = plsc.VectorSubcoreMesh(
        core_axis_name="c", subcore_axis_name="s", num_cores=_NC,
        num_subcores=_NS)

    out_sds = jax.ShapeDtypeStruct((BN * T,), jnp.float32)

    @functools.partial(
        pl.kernel,
        out_type=(out_sds, out_sds, out_sds),
        mesh=mesh,
        compiler_params=pltpu.CompilerParams(needs_layout_passes=False),
        scratch_types=[
            pltpu.VMEM((3 * N,), jnp.float32),
            pltpu.VMEM((N,), jnp.float32),
            pltpu.VMEM((N,), jnp.float32),
            pltpu.VMEM((N,), jnp.float32),
            pltpu.VMEM((CE,), jnp.int32),
            pltpu.VMEM((CE,), jnp.int32),
            pltpu.VMEM((CE,), jnp.int32),
            pltpu.VMEM((CE,), jnp.int32),
            pltpu.VMEM((CE,), jnp.float32),
            pltpu.VMEM((CE,), jnp.float32),
            pltpu.VMEM((CE,), jnp.float32),
            pltpu.VMEM((CE,), jnp.float32),
            pltpu.VMEM((CE,), jnp.float32),
            pltpu.VMEM((CE,), jnp.float32),
            pltpu.SemaphoreType.DMA,
            pltpu.SemaphoreType.DMA,
            pltpu.SemaphoreType.DMA,
            pltpu.SemaphoreType.DMA,
            pltpu.SemaphoreType.DMA,
        ],
    )
    def sc_kernel(pos_hbm, nj_hbm, nk_hbm,
                  rij_hbm, rik_hbm, rjk_hbm,
                  pv, xv, yv, zv, nj_a, nj_b, nk_a, nk_b,
                  oij_a, oij_b, oik_a, oik_b, ojk_a, ojk_b,
                  sem_pos, sem_in_a, sem_in_b, sem_out_a, sem_out_b):
        wid = lax.axis_index("s") * _NC + lax.axis_index("c")
        base_row = wid * rows_per
        batch_base = (base_row // N) * N
        base_local = base_row - batch_base

        bufs = {
            0: (nj_a, nk_a, oij_a, oik_a, ojk_a, sem_in_a, sem_out_a),
            1: (nj_b, nk_b, oij_b, oik_b, ojk_b, sem_in_b, sem_out_b),
        }

        def start_in(c, p):
            njx, nkx, _, _, _, sem, _ = bufs[p]
            off = (base_row + c * CH) * T
            pltpu.async_copy(nj_hbm.at[pl.ds(off, CE)], njx, sem)
            pltpu.async_copy(nk_hbm.at[pl.ds(off, CE)], nkx, sem)

        def wait_in(c, p):
            njx, nkx, _, _, _, sem, _ = bufs[p]
            off = (base_row + c * CH) * T
            pltpu.make_async_copy(nj_hbm.at[pl.ds(off, CE)], njx, sem).wait()
            pltpu.make_async_copy(nk_hbm.at[pl.ds(off, CE)], nkx, sem).wait()

        def start_out(c, p):
            _, _, oij, oik, ojk, _, sem = bufs[p]
            off = (base_row + c * CH) * T
            pltpu.async_copy(oij, rij_hbm.at[pl.ds(off, CE)], sem)
            pltpu.async_copy(oik, rik_hbm.at[pl.ds(off, CE)], sem)
            pltpu.async_copy(ojk, rjk_hbm.at[pl.ds(off, CE)], sem)

        def wait_out(c, p):
            _, _, oij, oik, ojk, _, sem = bufs[p]
            off = (base_row + c * CH) * T
            pltpu.make_async_copy(oij, rij_hbm.at[pl.ds(off, CE)], sem).wait()
            pltpu.make_async_copy(oik, rik_hbm.at[pl.ds(off, CE)], sem).wait()
            pltpu.make_async_copy(ojk, rjk_hbm.at[pl.ds(off, CE)], sem).wait()

        def compute(c, p):
            njx, nkx, oij, oik, ojk, _, _ = bufs[p]
            local0 = base_local + c * CH

            @plsc.parallel_loop(0, CH, step=1, unroll=1)
            def row_body(r):
                row_splat = jnp.full((_L,), local0 + r, jnp.int32)
                pk = lambda a, b: plsc.pack(
                    a, b, format=plsc.PackFormat.INTERLEAVED)
                xi = plsc.load_gather(xv, [row_splat])
                yi = plsc.load_gather(yv, [row_splat])
                zi = plsc.load_gather(zv, [row_splat])
                xib = pk(xi, xi)
                yib = pk(yi, yi)
                zib = pk(zi, zi)
                for v in range(0, T // _L, 2):
                    sl0 = pl.ds(r * T + v * _L, _L)
                    sl1 = pl.ds(r * T + (v + 1) * _L, _L)
                    g = []
                    for sl in (sl0, sl1):
                        j = njx[sl]
                        k = nkx[sl]
                        g.append((plsc.load_gather(xv, [j]),
                                  plsc.load_gather(yv, [j]),
                                  plsc.load_gather(zv, [j]),
                                  plsc.load_gather(xv, [k]),
                                  plsc.load_gather(yv, [k]),
                                  plsc.load_gather(zv, [k])))
                    xjb, yjb, zjb, xkb, ykb, zkb = (
                        pk(a, b) for a, b in zip(g[0], g[1]))
                    dxij = xjb - xib
                    dyij = yjb - yib
                    dzij = zjb - zib
                    dxik = xkb - xib
                    dyik = ykb - yib
                    dzik = zkb - zib
                    dxjk = xjb - xkb
                    dyjk = yjb - ykb
                    dzjk = zjb - zkb
                    d2ij = dxij * dxij + dyij * dyij + dzij * dzij
                    d2ik = dxik * dxik + dyik * dyik + dzik * dzik
                    d2jk = dxjk * dxjk + dyjk * dyjk + dzjk * dzjk
                    for out, d2 in ((oij, d2ij), (oik, d2ik), (ojk, d2jk)):
                        r0, r1 = plsc.unpack(_dist_bf16(d2),
                                             format=plsc.PackFormat.INTERLEAVED)
                        out[sl0] = r0
                        out[sl1] = r1

        pos_copy = pltpu.async_copy(
            pos_hbm.at[pl.ds(3 * batch_base, 3 * N)], pv, sem_pos)
        start_in(0, 0)
        pos_copy.wait()
        lane3 = lax.iota(jnp.int32, _L) * 3

        @plsc.parallel_loop(0, N // _L, step=1)
        def stage_body(i):
            sl = pl.ds(i * _L, _L)
            idx = lane3 + (i * (3 * _L))
            xv[sl] = plsc.load_gather(pv, [idx])
            yv[sl] = plsc.load_gather(pv, [idx + 1])
            zv[sl] = plsc.load_gather(pv, [idx + 2])

        def pair_body(c2, _):
            ca = 2 * c2
            cb = ca + 1
            start_in(cb, 1)
            wait_in(ca, 0)

            @pl.when(c2 > 0)
            def _():
                wait_out(ca - 2, 0)

            compute(ca, 0)
            start_out(ca, 0)

            @pl.when(c2 + 1 < pairs)
            def _():
                start_in(ca + 2, 0)

            wait_in(cb, 1)

            @pl.when(c2 > 0)
            def _():
                wait_out(cb - 2, 1)

            compute(cb, 1)
            start_out(cb, 1)
            return 0

        lax.fori_loop(0, pairs, pair_body, 0)
        wait_out(n_chunks - 2, 0)
        wait_out(n_chunks - 1, 1)

    return sc_kernel


def kernel(positions, neighbors_j, neighbors_k):
    B, N, _ = positions.shape
    T = neighbors_j.shape[2]
    BN = B * N

    pos = positions.reshape(BN * 3)
    nj = neighbors_j.reshape(BN * T)
    nk = neighbors_k.reshape(BN * T)

    rij, rik, rjk = _make_sc_kernel(B, N, T)(pos, nj, nk)
    shape = (B, N, T)
    return (rij.reshape(shape), rik.reshape(shape), rjk.reshape(shape))


# speedup vs baseline: 1.1307x; 1.1307x over previous
import functools

import jax
import jax.numpy as jnp
from jax import lax
from jax.experimental import pallas as pl
from jax.experimental.pallas import tpu as pltpu
from jax.experimental.pallas import tpu_sc as plsc

_NC = 2
_NS = 16
_NW = _NC * _NS
_L = 16

_MAGIC2 = 0x5F375F37


def _dist_bf16(d2):
    i = plsc.bitcast(d2, jnp.int32)
    seed = jnp.full((_L,), _MAGIC2, jnp.int32) - ((i >> 1) & 0x7FFF7FFF)
    y = plsc.bitcast(seed, jnp.bfloat16)
    t = d2 * y
    return t * (1.5 - 0.5 * (t * y))


def _make_sc_kernel(B, N, T):
    BN = B * N
    rows_per = BN // _NW
    CH = 64
    n_chunks = rows_per // CH
    pairs = n_chunks // 2
    CE = CH * T

    mesh = plsc.VectorSubcoreMesh(
        core_axis_name="c", subcore_axis_name="s", num_cores=_NC,
        num_subcores=_NS)

    out_sds = jax.ShapeDtypeStruct((BN * T,), jnp.float32)

    @functools.partial(
        pl.kernel,
        out_type=(out_sds, out_sds, out_sds),
        mesh=mesh,
        compiler_params=pltpu.CompilerParams(needs_layout_passes=False),
        scratch_types=[
            pltpu.VMEM((N,), jnp.float32),
            pltpu.VMEM((N,), jnp.float32),
            pltpu.VMEM((N,), jnp.float32),
            pltpu.VMEM((CE,), jnp.int32),
            pltpu.VMEM((CE,), jnp.int32),
            pltpu.VMEM((CE,), jnp.int32),
            pltpu.VMEM((CE,), jnp.int32),
            pltpu.VMEM((CE,), jnp.float32),
            pltpu.VMEM((CE,), jnp.float32),
            pltpu.VMEM((CE,), jnp.float32),
            pltpu.VMEM((CE,), jnp.float32),
            pltpu.VMEM((CE,), jnp.float32),
            pltpu.VMEM((CE,), jnp.float32),
            pltpu.SemaphoreType.DMA,
            pltpu.SemaphoreType.DMA,
            pltpu.SemaphoreType.DMA,
            pltpu.SemaphoreType.DMA,
        ],
    )
    def sc_kernel(x_hbm, y_hbm, z_hbm, nj_hbm, nk_hbm,
                  rij_hbm, rik_hbm, rjk_hbm,
                  xv, yv, zv, nj_a, nj_b, nk_a, nk_b,
                  oij_a, oij_b, oik_a, oik_b, ojk_a, ojk_b,
                  sem_in_a, sem_in_b, sem_out_a, sem_out_b):
        wid = lax.axis_index("s") * _NC + lax.axis_index("c")
        base_row = wid * rows_per
        batch_base = (base_row // N) * N
        base_local = base_row - batch_base

        bufs = {
            0: (nj_a, nk_a, oij_a, oik_a, ojk_a, sem_in_a, sem_out_a),
            1: (nj_b, nk_b, oij_b, oik_b, ojk_b, sem_in_b, sem_out_b),
        }

        def start_in(c, p):
            njx, nkx, _, _, _, sem, _ = bufs[p]
            off = (base_row + c * CH) * T
            pltpu.async_copy(nj_hbm.at[pl.ds(off, CE)], njx, sem)
            pltpu.async_copy(nk_hbm.at[pl.ds(off, CE)], nkx, sem)

        def wait_in(c, p):
            njx, nkx, _, _, _, sem, _ = bufs[p]
            off = (base_row + c * CH) * T
            pltpu.make_async_copy(nj_hbm.at[pl.ds(off, CE)], njx, sem).wait()
            pltpu.make_async_copy(nk_hbm.at[pl.ds(off, CE)], nkx, sem).wait()

        def start_out(c, p):
            _, _, oij, oik, ojk, _, sem = bufs[p]
            off = (base_row + c * CH) * T
            pltpu.async_copy(oij, rij_hbm.at[pl.ds(off, CE)], sem)
            pltpu.async_copy(oik, rik_hbm.at[pl.ds(off, CE)], sem)
            pltpu.async_copy(ojk, rjk_hbm.at[pl.ds(off, CE)], sem)

        def wait_out(c, p):
            _, _, oij, oik, ojk, _, sem = bufs[p]
            off = (base_row + c * CH) * T
            pltpu.make_async_copy(oij, rij_hbm.at[pl.ds(off, CE)], sem).wait()
            pltpu.make_async_copy(oik, rik_hbm.at[pl.ds(off, CE)], sem).wait()
            pltpu.make_async_copy(ojk, rjk_hbm.at[pl.ds(off, CE)], sem).wait()

        def compute(c, p):
            njx, nkx, oij, oik, ojk, _, _ = bufs[p]
            local0 = base_local + c * CH

            @plsc.parallel_loop(0, CH, step=1, unroll=1)
            def row_body(r):
                row_splat = jnp.full((_L,), local0 + r, jnp.int32)
                pk = lambda a, b: plsc.pack(
                    a, b, format=plsc.PackFormat.INTERLEAVED)
                xi = plsc.load_gather(xv, [row_splat])
                yi = plsc.load_gather(yv, [row_splat])
                zi = plsc.load_gather(zv, [row_splat])
                xib = pk(xi, xi)
                yib = pk(yi, yi)
                zib = pk(zi, zi)
                for v in range(0, T // _L, 2):
                    sl0 = pl.ds(r * T + v * _L, _L)
                    sl1 = pl.ds(r * T + (v + 1) * _L, _L)
                    g = []
                    for sl in (sl0, sl1):
                        j = njx[sl]
                        k = nkx[sl]
                        g.append((plsc.load_gather(xv, [j]),
                                  plsc.load_gather(yv, [j]),
                                  plsc.load_gather(zv, [j]),
                                  plsc.load_gather(xv, [k]),
                                  plsc.load_gather(yv, [k]),
                                  plsc.load_gather(zv, [k])))
                    xjb, yjb, zjb, xkb, ykb, zkb = (
                        pk(a, b) for a, b in zip(g[0], g[1]))
                    dxij = xjb - xib
                    dyij = yjb - yib
                    dzij = zjb - zib
                    dxik = xkb - xib
                    dyik = ykb - yib
                    dzik = zkb - zib
                    dxjk = xjb - xkb
                    dyjk = yjb - ykb
                    dzjk = zjb - zkb
                    d2ij = dxij * dxij + dyij * dyij + dzij * dzij
                    d2ik = dxik * dxik + dyik * dyik + dzik * dzik
                    d2jk = dxjk * dxjk + dyjk * dyjk + dzjk * dzjk
                    for out, d2 in ((oij, d2ij), (oik, d2ik), (ojk, d2jk)):
                        r0, r1 = plsc.unpack(_dist_bf16(d2),
                                             format=plsc.PackFormat.INTERLEAVED)
                        out[sl0] = r0
                        out[sl1] = r1

        pltpu.sync_copy(x_hbm.at[pl.ds(batch_base, N)], xv)
        pltpu.sync_copy(y_hbm.at[pl.ds(batch_base, N)], yv)
        pltpu.sync_copy(z_hbm.at[pl.ds(batch_base, N)], zv)

        start_in(0, 0)

        def pair_body(c2, _):
            ca = 2 * c2
            cb = ca + 1
            start_in(cb, 1)
            wait_in(ca, 0)

            @pl.when(c2 > 0)
            def _():
                wait_out(ca - 2, 0)

            compute(ca, 0)
            start_out(ca, 0)

            @pl.when(c2 + 1 < pairs)
            def _():
                start_in(ca + 2, 0)

            wait_in(cb, 1)

            @pl.when(c2 > 0)
            def _():
                wait_out(cb - 2, 1)

            compute(cb, 1)
            start_out(cb, 1)
            return 0

        lax.fori_loop(0, pairs, pair_body, 0)
        wait_out(n_chunks - 2, 0)
        wait_out(n_chunks - 1, 1)

    return sc_kernel


def kernel(positions, neighbors_j, neighbors_k):
    B, N, _ = positions.shape
    T = neighbors_j.shape[2]
    BN = B * N

    flat = positions.reshape(BN, 3)
    x = flat[:, 0].ravel()
    y = flat[:, 1].ravel()
    z = flat[:, 2].ravel()
    nj = neighbors_j.reshape(BN * T)
    nk = neighbors_k.reshape(BN * T)

    rij, rik, rjk = _make_sc_kernel(B, N, T)(x, y, z, nj, nk)
    shape = (B, N, T)
    return (rij.reshape(shape), rik.reshape(shape), rjk.reshape(shape))
